# baseline (speedup 1.0000x reference)
import jax
import jax.numpy as jnp
from jax.experimental import pallas as pl
from jax.experimental.pallas import tpu as pltpu

_DIM = 768
_H = 12
_K = 90
_B = 8
_N = 576
_HD = _DIM // _H
_SCALE = _HD ** -0.5
_MININT = -(2 ** 31)


def _key2f(t):
    st = t ^ _MININT
    fb = st ^ ((st >> 31) & 0x7FFFFFFF)
    return jax.lax.bitcast_convert_type(fb, jnp.float32)


def _fused_kernel(islast_ref, x_ref, wqkv_ref, wpt_ref, bias_ref,
                  attn_ref, out_ref):
    h = pl.program_id(1)
    x = x_ref[0]
    wq = wqkv_ref[pl.ds(h * _HD, _HD), :]
    wk = wqkv_ref[pl.ds((_H + h) * _HD, _HD), :]
    wv = wqkv_ref[pl.ds((2 * _H + h) * _HD, _HD), :]
    cdims = (((1,), (1,)), ((), ()))
    q = jax.lax.dot_general(x, wq, cdims, preferred_element_type=jnp.float32)
    k = jax.lax.dot_general(x, wk, cdims, preferred_element_type=jnp.float32)
    v = jax.lax.dot_general(x, wv, cdims, preferred_element_type=jnp.float32)
    st = jax.lax.dot_general(
        k, q, cdims, preferred_element_type=jnp.float32) * _SCALE

    bits = jax.lax.bitcast_convert_type(st, jnp.int32)
    su = bits ^ ((bits >> 31) & 0x7FFFFFFF)
    kb = (su >> 16).astype(jnp.int16)

    def body_hi(i, T):
        b = 31 - i
        cand = T + (jnp.int32(1) << b)
        ch = (cand >> 16).astype(jnp.int16)
        c = jnp.sum((kb >= ch).astype(jnp.int32), axis=0, keepdims=True)
        return jnp.where(c >= _K, cand, T)

    def body_lo(i, T):
        b = 15 - i
        cand = T + (jnp.int32(1) << b)
        fb = cand ^ ((cand >> 31) & 0x7FFFFFFF)
        thrf = jax.lax.bitcast_convert_type(fb, jnp.float32)
        c = jnp.sum((st >= thrf).astype(jnp.int32), axis=0, keepdims=True)
        return jnp.where(c >= _K, cand, T)

    T0 = jnp.full((1, _N), _MININT, jnp.int32)
    T = jax.lax.fori_loop(0, 16, body_hi, T0)
    T = jax.lax.fori_loop(0, 16, body_lo, T)

    fbT = T ^ ((T >> 31) & 0x7FFFFFFF)
    thr = jax.lax.bitcast_convert_type(fbT, jnp.float32)
    thr = jnp.where(islast_ref[0] == 0, thr, -jnp.inf)
    mask = st >= thr

    m = jnp.max(st, axis=0, keepdims=True)
    p = jnp.where(mask, jnp.exp(st - m), 0.0)
    a = p / jnp.sum(p, axis=0, keepdims=True)
    attn_ref[0, 0] = a.T

    av = jax.lax.dot_general(
        a, v, dimension_numbers=(((0,), (0,)), ((), ())),
        preferred_element_type=jnp.float32)
    wpt = wpt_ref[pl.ds(h * _HD, _HD), :]
    contrib = jax.lax.dot_general(
        av, wpt, dimension_numbers=(((1,), (0,)), ((), ())),
        preferred_element_type=jnp.float32)

    @pl.when(h == 0)
    def _init():
        out_ref[0] = contrib + bias_ref[...]

    @pl.when(h != 0)
    def _acc():
        out_ref[0] += contrib


def kernel(x, W_qkv, W_proj, b_proj, islast):
    islast_arr = jnp.asarray(islast, jnp.int32).reshape(1)

    attn, out = pl.pallas_call(
        _fused_kernel,
        grid=(_B, _H),
        in_specs=[
            pl.BlockSpec(memory_space=pltpu.SMEM),
            pl.BlockSpec((1, _N, _DIM), lambda b, h: (b, 0, 0)),
            pl.BlockSpec((3 * _DIM, _DIM), lambda b, h: (0, 0)),
            pl.BlockSpec((_DIM, _DIM), lambda b, h: (0, 0)),
            pl.BlockSpec((1, _DIM), lambda b, h: (0, 0)),
        ],
        out_specs=[
            pl.BlockSpec((1, 1, _N, _N), lambda b, h: (b, h, 0, 0)),
            pl.BlockSpec((1, _N, _DIM), lambda b, h: (b, 0, 0)),
        ],
        out_shape=[
            jax.ShapeDtypeStruct((_B, _H, _N, _N), jnp.float32),
            jax.ShapeDtypeStruct((_B, _N, _DIM), jnp.float32),
        ],
        compiler_params=pltpu.CompilerParams(
            dimension_semantics=("parallel", "arbitrary")),
    )(islast_arr, x, W_qkv, W_proj.T, b_proj.reshape(1, _DIM))

    return (out, attn)


# speedup vs baseline: 1.1565x; 1.1565x over previous
import jax
import jax.numpy as jnp
from jax.experimental import pallas as pl
from jax.experimental.pallas import tpu as pltpu

_DIM = 768
_H = 12
_K = 90
_B = 8
_N = 576
_HD = _DIM // _H
_SCALE = _HD ** -0.5
_MININT = -(2 ** 31)


def _key2f(t):
    st = t ^ _MININT
    fb = st ^ ((st >> 31) & 0x7FFFFFFF)
    return jax.lax.bitcast_convert_type(fb, jnp.float32)


def _fused_kernel(islast_ref, x_ref, wqkv_ref, wpt_ref, bias_ref,
                  attn_ref, out_ref):
    h = pl.program_id(1)
    x = x_ref[0]
    wq = wqkv_ref[pl.ds(h * _HD, _HD), :]
    wk = wqkv_ref[pl.ds((_H + h) * _HD, _HD), :]
    wv = wqkv_ref[pl.ds((2 * _H + h) * _HD, _HD), :]
    cdims = (((1,), (1,)), ((), ()))
    q = jax.lax.dot_general(x, wq, cdims, preferred_element_type=jnp.float32)
    k = jax.lax.dot_general(x, wk, cdims, preferred_element_type=jnp.float32)
    v = jax.lax.dot_general(x, wv, cdims, preferred_element_type=jnp.float32)
    st = jax.lax.dot_general(
        k, q, cdims, preferred_element_type=jnp.float32) * _SCALE

    def cond(carry):
        i, T, candlock, locked, alldone = carry
        return jnp.logical_and(i < 32, alldone == 0)

    def body(carry):
        i, T, candlock, locked, alldone = carry
        cand = T | (jnp.int32(1) << (31 - i))
        thrf = _key2f(cand)
        c = jnp.sum((st >= thrf).astype(jnp.int32), axis=0, keepdims=True)
        T = jnp.where(c >= _K, cand, T)
        hit = (c == _K).astype(jnp.int32)
        candlock = jnp.where((locked == 0) & (hit == 1), cand, candlock)
        locked = jnp.maximum(locked, hit)
        alldone = jnp.min(locked)
        return (i + 1, T, candlock, locked, alldone)

    z = jnp.zeros((1, _N), jnp.int32)
    (_, T, candlock, locked, _) = jax.lax.while_loop(
        cond, body, (jnp.int32(0), z, z, z, jnp.int32(0)))

    seed = jnp.where(locked == 1, _key2f(candlock), _key2f(T))
    thr = jnp.min(jnp.where(st >= seed, st, jnp.inf), axis=0, keepdims=True)
    thr = jnp.where(islast_ref[0] == 0, thr, -jnp.inf)
    mask = st >= thr

    m = jnp.max(st, axis=0, keepdims=True)
    p = jnp.where(mask, jnp.exp(st - m), 0.0)
    a = p / jnp.sum(p, axis=0, keepdims=True)
    attn_ref[0, 0] = a.T

    av = jax.lax.dot_general(
        a, v, dimension_numbers=(((0,), (0,)), ((), ())),
        preferred_element_type=jnp.float32)
    wpt = wpt_ref[pl.ds(h * _HD, _HD), :]
    contrib = jax.lax.dot_general(
        av, wpt, dimension_numbers=(((1,), (0,)), ((), ())),
        preferred_element_type=jnp.float32)

    @pl.when(h == 0)
    def _init():
        out_ref[0] = contrib + bias_ref[...]

    @pl.when(h != 0)
    def _acc():
        out_ref[0] += contrib


def kernel(x, W_qkv, W_proj, b_proj, islast):
    islast_arr = jnp.asarray(islast, jnp.int32).reshape(1)

    attn, out = pl.pallas_call(
        _fused_kernel,
        grid=(_B, _H),
        in_specs=[
            pl.BlockSpec(memory_space=pltpu.SMEM),
            pl.BlockSpec((1, _N, _DIM), lambda b, h: (b, 0, 0)),
            pl.BlockSpec((3 * _DIM, _DIM), lambda b, h: (0, 0)),
            pl.BlockSpec((_DIM, _DIM), lambda b, h: (0, 0)),
            pl.BlockSpec((1, _DIM), lambda b, h: (0, 0)),
        ],
        out_specs=[
            pl.BlockSpec((1, 1, _N, _N), lambda b, h: (b, h, 0, 0)),
            pl.BlockSpec((1, _N, _DIM), lambda b, h: (b, 0, 0)),
        ],
        out_shape=[
            jax.ShapeDtypeStruct((_B, _H, _N, _N), jnp.float32),
            jax.ShapeDtypeStruct((_B, _N, _DIM), jnp.float32),
        ],
        compiler_params=pltpu.CompilerParams(
            dimension_semantics=("parallel", "arbitrary")),
    )(islast_arr, x, W_qkv, W_proj.T, b_proj.reshape(1, _DIM))

    return (out, attn)


# speedup vs baseline: 1.7351x; 1.5003x over previous
import jax
import jax.numpy as jnp
from jax.experimental import pallas as pl
from jax.experimental.pallas import tpu as pltpu

_DIM = 768
_H = 12
_K = 90
_B = 8
_N = 576
_HD = _DIM // _H
_SCALE = _HD ** -0.5
_MININT = -(2 ** 31)


def _key2f(t):
    st = t ^ _MININT
    fb = st ^ ((st >> 31) & 0x7FFFFFFF)
    return jax.lax.bitcast_convert_type(fb, jnp.float32)


def _fused_kernel(islast_ref, x_ref, wqkv_ref, wpt_ref, bias_ref,
                  attn_ref, out_ref):
    h = pl.program_id(1)
    x = x_ref[0]
    wq = wqkv_ref[pl.ds(h * _HD, _HD), :]
    wk = wqkv_ref[pl.ds((_H + h) * _HD, _HD), :]
    wv = wqkv_ref[pl.ds((2 * _H + h) * _HD, _HD), :]
    cdims = (((1,), (1,)), ((), ()))
    q = jax.lax.dot_general(x, wq, cdims, preferred_element_type=jnp.float32)
    k = jax.lax.dot_general(x, wk, cdims, preferred_element_type=jnp.float32)
    v = jax.lax.dot_general(x, wv, cdims, preferred_element_type=jnp.float32)
    st = jax.lax.dot_general(
        k, q, cdims, preferred_element_type=jnp.float32) * _SCALE

    def body(i, T):
        b = 31 - i
        cand = T | (jnp.int32(1) << b)
        thrf = _key2f(cand)
        c = jnp.sum((st >= thrf).astype(jnp.float32), axis=0,
                    keepdims=True)
        return jnp.where(c >= float(_K), cand, T)

    T = jax.lax.fori_loop(0, 32, body, jnp.zeros((1, _N), jnp.int32))

    thr = _key2f(T)
    thr = jnp.where(islast_ref[0] == 0, thr, -jnp.inf)
    mask = st >= thr

    m = jnp.max(st, axis=0, keepdims=True)
    p = jnp.where(mask, jnp.exp(st - m), 0.0)
    a = p / jnp.sum(p, axis=0, keepdims=True)
    attn_ref[0, 0] = a.T

    av = jax.lax.dot_general(
        a, v, dimension_numbers=(((0,), (0,)), ((), ())),
        preferred_element_type=jnp.float32)
    wpt = wpt_ref[pl.ds(h * _HD, _HD), :]
    contrib = jax.lax.dot_general(
        av, wpt, dimension_numbers=(((1,), (0,)), ((), ())),
        preferred_element_type=jnp.float32)

    @pl.when(h == 0)
    def _init():
        out_ref[0] = contrib + bias_ref[...]

    @pl.when(h != 0)
    def _acc():
        out_ref[0] += contrib


def kernel(x, W_qkv, W_proj, b_proj, islast):
    islast_arr = jnp.asarray(islast, jnp.int32).reshape(1)

    attn, out = pl.pallas_call(
        _fused_kernel,
        grid=(_B, _H),
        in_specs=[
            pl.BlockSpec(memory_space=pltpu.SMEM),
            pl.BlockSpec((1, _N, _DIM), lambda b, h: (b, 0, 0)),
            pl.BlockSpec((3 * _DIM, _DIM), lambda b, h: (0, 0)),
            pl.BlockSpec((_DIM, _DIM), lambda b, h: (0, 0)),
            pl.BlockSpec((1, _DIM), lambda b, h: (0, 0)),
        ],
        out_specs=[
            pl.BlockSpec((1, 1, _N, _N), lambda b, h: (b, h, 0, 0)),
            pl.BlockSpec((1, _N, _DIM), lambda b, h: (b, 0, 0)),
        ],
        out_shape=[
            jax.ShapeDtypeStruct((_B, _H, _N, _N), jnp.float32),
            jax.ShapeDtypeStruct((_B, _N, _DIM), jnp.float32),
        ],
        compiler_params=pltpu.CompilerParams(
            dimension_semantics=("parallel", "arbitrary")),
    )(islast_arr, x, W_qkv, W_proj.T, b_proj.reshape(1, _DIM))

    return (out, attn)
